# baseline (speedup 1.0000x reference)
import functools

import jax
import jax.numpy as jnp
from jax.experimental import pallas as pl
from jax.experimental.pallas import tpu as pltpu
from jax.experimental.pallas import tpu_sc as plsc

_EPS = 1e-5


def _pre1_body(x_ref, w1t_ref, h1_ref, st_ref):
    i = pl.program_id(0)
    h1 = jnp.dot(x_ref[...], w1t_ref[...], preferred_element_type=jnp.float32)
    h1_ref[...] = h1

    @pl.when(i == 0)
    def _():
        st_ref[...] = jnp.zeros_like(st_ref)

    s = jnp.sum(h1, axis=0)
    ss = jnp.sum(h1 * h1, axis=0)
    st_ref[...] += jnp.concatenate([s[None, :], ss[None, :]], axis=0)


def _pre2_body(h1_ref, aff_ref, wqkv_ref, bqkv_ref, q_ref, k_ref, v_ref):
    h = jnp.maximum(h1_ref[...] * aff_ref[0:1, :] + aff_ref[1:2, :], 0.0)
    qkv = jnp.dot(h, wqkv_ref[...], preferred_element_type=jnp.float32)
    qkv = qkv + bqkv_ref[...]
    C = q_ref.shape[1]
    q_ref[...] = qkv[:, :C]
    k_ref[...] = qkv[:, C:2 * C]
    v_ref[...] = qkv[:, 2 * C:]


def _knn_body(pq_ref, pt_ref, ind_ref, d_ref, nsel):
    tq = pq_ref.shape[0]
    pq = pq_ref[...]
    pt = pt_ref[...]
    n = pt.shape[1]
    dot = jnp.dot(pq, pt, preferred_element_type=jnp.float32)
    sqk = jnp.sum(pt * pt, axis=0, keepdims=True)
    sqq = jnp.sum(pq * pq, axis=1, keepdims=True)
    iota_f = jax.lax.broadcasted_iota(jnp.int32, (tq, n), 1).astype(jnp.float32)
    d = sqq + sqk - 2.0 * dot
    big = jnp.float32(jnp.inf)
    nf = jnp.float32(n)
    cols = []
    idxf = None
    for k in range(nsel):
        if k == 0:
            d_ref[...] = d
            v = d
        else:
            v = jnp.where(iota_f == idxf, big, d_ref[...])
            d_ref[...] = v
        m = jnp.min(v, axis=1, keepdims=True)
        idxf = jnp.min(jnp.where(v == m, iota_f, nf), axis=1, keepdims=True)
        cols.append(idxf)
    pad = jnp.zeros((tq, 32 - nsel), jnp.float32)
    ind_ref[...] = jnp.concatenate(cols + [pad], axis=1).astype(jnp.int32)


def _p0_body(pg_ref, p8_ref, ew_ref, w1p_ref, b1p_ref, pr3_ref, st_ref):
    i = pl.program_id(0)
    r, c8 = p8_ref.shape
    k = pg_ref.shape[0] // r
    prj = jnp.dot(pg_ref[...], ew_ref[...],
                  preferred_element_type=jnp.float32)
    pri = jnp.dot(p8_ref[...], w1p_ref[...],
                  preferred_element_type=jnp.float32)
    pr = (prj.reshape(r, k, c8) - pri.reshape(r, 1, c8)
          + b1p_ref[...].reshape(1, 1, c8)).reshape(r * k, c8)
    pr3_ref[...] = pr

    @pl.when(i == 0)
    def _():
        st_ref[...] = jnp.zeros_like(st_ref)

    s = jnp.sum(pr, axis=0)
    ss = jnp.sum(pr * pr, axis=0)
    st_ref[...] += jnp.concatenate([s[None, :], ss[None, :]], axis=0)


def _pr_block(pr3, a3, wp2, bp2, r, k):
    pr3n = jnp.maximum(pr3 * a3[0:1, :] + a3[1:2, :], 0.0)
    pr = jnp.dot(pr3n.astype(jnp.bfloat16), wp2,
                 preferred_element_type=jnp.float32) + bp2
    return pr.reshape(r, k, -1)


def _p1_body(q_ref, kg_ref, pr3_ref, a3_ref, wp2_ref, bp2_ref, st_ref):
    i = pl.program_id(0)
    r, k, c = kg_ref.shape
    pr = _pr_block(pr3_ref[...], a3_ref[...], wp2_ref[...], bp2_ref[...],
                   r, k)
    w = q_ref[...].reshape(r, 1, c) - kg_ref[...] + pr

    @pl.when(i == 0)
    def _():
        st_ref[...] = jnp.zeros_like(st_ref)

    s = jnp.sum(w, axis=(0, 1))
    ss = jnp.sum(w * w, axis=(0, 1))
    st_ref[...] += jnp.concatenate([s[None, :], ss[None, :]], axis=0)


def _p2_body(q_ref, kg_ref, pr3_ref, a3_ref, wp2_ref, bp2_ref,
             a1_ref, ww1_ref, bw1_ref, u_ref, st_ref):
    i = pl.program_id(0)
    r, k, c = kg_ref.shape
    pr = _pr_block(pr3_ref[...], a3_ref[...], wp2_ref[...], bp2_ref[...],
                   r, k)
    w = q_ref[...].reshape(r, 1, c) - kg_ref[...] + pr
    w = jnp.maximum(w * a1_ref[0:1, :].reshape(1, 1, c)
                    + a1_ref[1:2, :].reshape(1, 1, c), 0.0)
    u = jnp.dot(w.reshape(r * k, c).astype(jnp.bfloat16), ww1_ref[...],
                preferred_element_type=jnp.float32) + bw1_ref[...]
    u_ref[...] = u

    @pl.when(i == 0)
    def _():
        st_ref[...] = jnp.zeros_like(st_ref)

    s = jnp.sum(u, axis=0)
    ss = jnp.sum(u * u, axis=0)
    st_ref[...] += jnp.concatenate([s[None, :], ss[None, :]], axis=0)


def _p3_body(u_ref, vg_ref, pr3_ref, a3_ref, wp2_ref, bp2_ref,
             a2_ref, ww2_ref, bw2_ref, t_ref, st_ref):
    i = pl.program_id(0)
    r, k, c = vg_ref.shape
    un = jnp.maximum(u_ref[...] * a2_ref[0:1, :] + a2_ref[1:2, :], 0.0)
    w2 = jnp.dot(un.astype(jnp.bfloat16), ww2_ref[...],
                 preferred_element_type=jnp.float32) + bw2_ref[...]
    w2 = w2.reshape(r, k, c)
    mx = jnp.max(w2, axis=1, keepdims=True)
    e = jnp.exp(w2 - mx)
    wgt = e / jnp.sum(e, axis=1, keepdims=True)
    pr = _pr_block(pr3_ref[...], a3_ref[...], wp2_ref[...], bp2_ref[...],
                   r, k)
    t = jnp.sum((vg_ref[...] + pr) * wgt, axis=1)
    t_ref[...] = t

    @pl.when(i == 0)
    def _():
        st_ref[...] = jnp.zeros_like(st_ref)

    s = jnp.sum(t, axis=0)
    ss = jnp.sum(t * t, axis=0)
    st_ref[...] += jnp.concatenate([s[None, :], ss[None, :]], axis=0)


def _p4_body(t_ref, at_ref, w3_ref, s_ref, st_ref):
    i = pl.program_id(0)
    tn = jnp.maximum(t_ref[...] * at_ref[0:1, :] + at_ref[1:2, :], 0.0)
    s = jnp.dot(tn.astype(jnp.bfloat16), w3_ref[...],
                preferred_element_type=jnp.float32)
    s_ref[...] = s

    @pl.when(i == 0)
    def _():
        st_ref[...] = jnp.zeros_like(st_ref)

    su = jnp.sum(s, axis=0)
    ss = jnp.sum(s * s, axis=0)
    st_ref[...] += jnp.concatenate([su[None, :], ss[None, :]], axis=0)


def _p5_body(s_ref, x_ref, as_ref, o_ref):
    o_ref[...] = jnp.maximum(
        s_ref[...] * as_ref[0:1, :] + as_ref[1:2, :] + x_ref[...], 0.0)


_SC_WINDOW = 128


def _sc_gather2(tab, pp, gi):
    m = gi.shape[0] * gi.shape[1]
    c = tab.shape[1]
    cp = pp.shape[1]
    w = _SC_WINDOW
    mesh = plsc.VectorSubcoreMesh(core_axis_name="core",
                                  subcore_axis_name="subcore")

    @functools.partial(
        pl.kernel,
        out_type=(jax.ShapeDtypeStruct((m, c), jnp.float32),
                  jax.ShapeDtypeStruct((m, cp), jnp.float32)),
        mesh=mesh,
        scratch_types=[],
    )
    def k(tab_hbm, pp_hbm, gi_hbm, tg_hbm, ppg_hbm):
        def body(i_vmem, to_vmem, ppo_vmem):
            pltpu.sync_copy(tab_hbm.at[i_vmem.at[0]], to_vmem)
            pltpu.sync_copy(pp_hbm.at[i_vmem.at[0]], ppo_vmem)

        pltpu.emit_pipeline(
            body,
            grid=(m // w,),
            in_specs=[pl.BlockSpec((1, w), lambda i: (i, 0))],
            out_specs=[pl.BlockSpec((w, c), lambda i: (i, 0)),
                       pl.BlockSpec((w, cp), lambda i: (i, 0))],
            core_axis_name=("core", "subcore"),
            dimension_semantics=(pltpu.PARALLEL,),
        )(gi_hbm, tg_hbm, ppg_hbm)

    return k(tab, pp, gi)


def _sc_gather1(tab, gi):
    m = gi.shape[0] * gi.shape[1]
    c = tab.shape[1]
    w = _SC_WINDOW
    mesh = plsc.VectorSubcoreMesh(core_axis_name="core",
                                  subcore_axis_name="subcore")

    @functools.partial(
        pl.kernel,
        out_type=jax.ShapeDtypeStruct((m, c), jnp.float32),
        mesh=mesh,
        scratch_types=[],
    )
    def k(tab_hbm, gi_hbm, tg_hbm):
        def body(i_vmem, to_vmem):
            pltpu.sync_copy(tab_hbm.at[i_vmem.at[0]], to_vmem)

        pltpu.emit_pipeline(
            body,
            grid=(m // w,),
            in_specs=[pl.BlockSpec((1, w), lambda i: (i, 0))],
            out_specs=[pl.BlockSpec((w, c), lambda i: (i, 0))],
            core_axis_name=("core", "subcore"),
            dimension_semantics=(pltpu.PARALLEL,),
        )(gi_hbm, tg_hbm)

    return k(tab, gi)


def _affine(st, g, b, count):
    m = st[0] / count
    var = jnp.maximum(st[1] / count - m * m, 0.0)
    scale = g / jnp.sqrt(var + _EPS)
    shift = b - m * scale
    return jnp.concatenate([scale[None, :], shift[None, :]], axis=0)


def kernel(x, p, knn_ind, params):
    B, N, C = x.shape
    K = knn_ind.shape[2]
    BN = B * N
    M = BN * K
    TM = 512
    TR = 64

    x2 = x.reshape(BN, C)
    p2 = p.reshape(BN, 3)
    p8 = jnp.concatenate([p2, jnp.zeros((BN, 5), jnp.float32)], axis=1)
    p128 = jnp.concatenate([p2, jnp.zeros((BN, 125), jnp.float32)], axis=1)
    pt = jnp.swapaxes(p, 1, 2)

    P = params
    w1t = P['W1'].T
    wqkv = jnp.concatenate([P['Wq'].T, P['Wk'].T, P['Wv'].T], axis=1)
    bqkv = jnp.concatenate([P['bq'], P['bk'], P['bv']])[None, :]
    w1p = jnp.zeros((8, 8), jnp.float32).at[:3, :3].set(P['Wp1'].T)
    ew = jnp.zeros((128, 8), jnp.float32).at[:3, :3].set(P['Wp1'].T)
    b1p = jnp.zeros((1, 8), jnp.float32).at[0, :3].set(P['bp1'])
    g3 = jnp.zeros((8,), jnp.float32).at[:3].set(P['bnp_g'])
    b3 = jnp.zeros((8,), jnp.float32).at[:3].set(P['bnp_b'])
    wp2 = jnp.zeros((8, C), jnp.float32).at[:3, :].set(
        P['Wp2'].T).astype(jnp.bfloat16)
    bp2 = P['bp2'][None, :]
    ww1 = P['Ww1'].T.astype(jnp.bfloat16)
    bw1 = P['bw1'][None, :]
    ww2 = P['Ww2'].T.astype(jnp.bfloat16)
    bw2 = P['bw2'][None, :]
    w3 = P['W3'].T.astype(jnp.bfloat16)

    cspec = pl.BlockSpec((2, C), lambda i: (0, 0))
    wspec = pl.BlockSpec((C, C), lambda i: (0, 0))
    rspec = pl.BlockSpec((TM, C), lambda i: (i, 0))

    h1, st1 = pl.pallas_call(
        _pre1_body,
        grid=(BN // TM,),
        in_specs=[rspec, wspec],
        out_specs=[rspec, cspec],
        out_shape=[jax.ShapeDtypeStruct((BN, C), jnp.float32),
                   jax.ShapeDtypeStruct((2, C), jnp.float32)],
    )(x2, w1t)
    aff1 = _affine(st1, P['bn1_g'], P['bn1_b'], BN)

    q, kt, vt = pl.pallas_call(
        _pre2_body,
        grid=(BN // TM,),
        in_specs=[rspec, cspec, pl.BlockSpec((C, 3 * C), lambda i: (0, 0)),
                  pl.BlockSpec((1, 3 * C), lambda i: (0, 0))],
        out_specs=[rspec, rspec, rspec],
        out_shape=[jax.ShapeDtypeStruct((BN, C), jnp.float32),
                   jax.ShapeDtypeStruct((BN, C), jnp.float32),
                   jax.ShapeDtypeStruct((BN, C), jnp.float32)],
    )(h1, aff1, wqkv, bqkv)

    NCPB = 2
    NCH = B * NCPB
    R = N // NCPB
    MC = R * K
    TQ = 256
    TP = 128

    chunks = []
    for c in range(NCH):
        b, off = c // NCPB, (c % NCPB) * R
        ind_c = pl.pallas_call(
            functools.partial(_knn_body, nsel=K + 1),
            grid=(R // TQ,),
            in_specs=[pl.BlockSpec((TQ, 3), lambda i: (i, 0)),
                      pl.BlockSpec((3, N), lambda i: (0, 0))],
            out_specs=pl.BlockSpec((TQ, 32), lambda i: (i, 0)),
            out_shape=jax.ShapeDtypeStruct((R, 32), jnp.int32),
            scratch_shapes=[pltpu.VMEM((TQ, N), jnp.float32)],
        )(p[b, off:off + R], pt[b])
        gi_c = (ind_c[:, 1:K + 1] + b * N).reshape(MC // _SC_WINDOW,
                                                   _SC_WINDOW)
        chunks.append((ind_c, gi_c))

    gath = []
    for c in range(NCH):
        _, gi_c = chunks[c]
        kg_c, pg_c = _sc_gather2(kt, p128, gi_c)
        gath.append((kg_c.reshape(R, K, C), pg_c))

    pr3s, st3s = [], []
    for c in range(NCH):
        base = (c // NCPB) * N + (c % NCPB) * R
        pr3_c, st3_c = pl.pallas_call(
            _p0_body,
            grid=(R // TP,),
            in_specs=[pl.BlockSpec((TP * K, 128), lambda i: (i, 0)),
                      pl.BlockSpec((TP, 8),
                                   functools.partial(
                                       lambda base, i: (i + base, 0),
                                       base // TP)),
                      pl.BlockSpec((128, 8), lambda i: (0, 0)),
                      pl.BlockSpec((8, 8), lambda i: (0, 0)),
                      pl.BlockSpec((1, 8), lambda i: (0, 0))],
            out_specs=[pl.BlockSpec((TP * K, 8), lambda i: (i, 0)),
                       pl.BlockSpec((2, 8), lambda i: (0, 0))],
            out_shape=[jax.ShapeDtypeStruct((MC, 8), jnp.float32),
                       jax.ShapeDtypeStruct((2, 8), jnp.float32)],
        )(gath[c][1], p8, ew, w1p, b1p)
        pr3s.append(pr3_c)
        st3s.append(st3_c)
    aff3 = _affine(sum(st3s[1:], st3s[0]), g3, b3, M)

    kgspec = pl.BlockSpec((TR, K, C), lambda i: (i, 0, 0))
    vgspec = pl.BlockSpec((TR, K, C), lambda i: (i, 0, 0))
    pr3spec = pl.BlockSpec((TR * K, 8), lambda i: (i, 0))
    a3spec = pl.BlockSpec((2, 8), lambda i: (0, 0))
    wp2spec = pl.BlockSpec((8, C), lambda i: (0, 0))
    bspec = pl.BlockSpec((1, C), lambda i: (0, 0))
    uspec = pl.BlockSpec((TR * K, C), lambda i: (i, 0))

    def _qspec(c):
        base = ((c // NCPB) * N + (c % NCPB) * R) // TR
        return pl.BlockSpec(
            (TR, C), functools.partial(lambda b0, i: (i + b0, 0), base))

    stw1 = None
    for c in range(NCH):
        st_c = pl.pallas_call(
            _p1_body,
            grid=(R // TR,),
            in_specs=[_qspec(c), kgspec, pr3spec, a3spec, wp2spec, bspec],
            out_specs=cspec,
            out_shape=jax.ShapeDtypeStruct((2, C), jnp.float32),
        )(q, gath[c][0], pr3s[c], aff3, wp2, bp2)
        stw1 = st_c if stw1 is None else stw1 + st_c
    affw1 = _affine(stw1, P['bnw1_g'], P['bnw1_b'], M)

    vgs = [_sc_gather1(vt, chunks[c][1]).reshape(R, K, C)
           for c in range(NCH)]

    us, stw2 = [], None
    for c in range(NCH):
        u_c, st_c = pl.pallas_call(
            _p2_body,
            grid=(R // TR,),
            in_specs=[_qspec(c), kgspec, pr3spec, a3spec, wp2spec, bspec,
                      cspec, wspec, bspec],
            out_specs=[uspec, cspec],
            out_shape=[jax.ShapeDtypeStruct((MC, C), jnp.float32),
                       jax.ShapeDtypeStruct((2, C), jnp.float32)],
        )(q, gath[c][0], pr3s[c], aff3, wp2, bp2, affw1, ww1, bw1)
        us.append(u_c)
        stw2 = st_c if stw2 is None else stw2 + st_c
    affw2 = _affine(stw2, P['bnw2_g'], P['bnw2_b'], M)

    ts, stt = [], None
    for c in range(NCH):
        t_c, st_c = pl.pallas_call(
            _p3_body,
            grid=(R // TR,),
            in_specs=[uspec, vgspec, pr3spec, a3spec, wp2spec, bspec,
                      cspec, wspec, bspec],
            out_specs=[pl.BlockSpec((TR, C), lambda i: (i, 0)), cspec],
            out_shape=[jax.ShapeDtypeStruct((R, C), jnp.float32),
                       jax.ShapeDtypeStruct((2, C), jnp.float32)],
        )(us[c], vgs[c], pr3s[c], aff3, wp2, bp2, affw2, ww2, bw2)
        ts.append(t_c)
        stt = st_c if stt is None else stt + st_c
    t = jnp.concatenate(ts, axis=0)
    afft = _affine(stt, P['bn2_g'], P['bn2_b'], BN)

    s, sts = pl.pallas_call(
        _p4_body,
        grid=(BN // TM,),
        in_specs=[rspec, cspec, wspec],
        out_specs=[rspec, cspec],
        out_shape=[jax.ShapeDtypeStruct((BN, C), jnp.float32),
                   jax.ShapeDtypeStruct((2, C), jnp.float32)],
    )(t, afft, w3)
    affs = _affine(sts, P['bn3_g'], P['bn3_b'], BN)

    out = pl.pallas_call(
        _p5_body,
        grid=(BN // TM,),
        in_specs=[rspec, rspec, cspec],
        out_specs=rspec,
        out_shape=jax.ShapeDtypeStruct((BN, C), jnp.float32),
    )(s, x2, affs)

    ind = jnp.stack(
        [jnp.concatenate([chunks[b * NCPB + j][0][:, 1:K + 1]
                          for j in range(NCPB)], axis=0) for b in range(B)],
        axis=0)
    return (out.reshape(B, N, C), p, ind)


# speedup vs baseline: 1.0882x; 1.0882x over previous
import functools

import jax
import jax.numpy as jnp
from jax.experimental import pallas as pl
from jax.experimental.pallas import tpu as pltpu
from jax.experimental.pallas import tpu_sc as plsc

_EPS = 1e-5


def _pre1_body(x_ref, w1t_ref, h1_ref, st_ref):
    i = pl.program_id(0)
    h1 = jnp.dot(x_ref[...], w1t_ref[...], preferred_element_type=jnp.float32)
    h1_ref[...] = h1

    @pl.when(i == 0)
    def _():
        st_ref[...] = jnp.zeros_like(st_ref)

    s = jnp.sum(h1, axis=0)
    ss = jnp.sum(h1 * h1, axis=0)
    st_ref[...] += jnp.concatenate([s[None, :], ss[None, :]], axis=0)


def _pre2_body(h1_ref, aff_ref, wqkv_ref, bqkv_ref, q_ref, k_ref, v_ref):
    h = jnp.maximum(h1_ref[...] * aff_ref[0:1, :] + aff_ref[1:2, :], 0.0)
    qkv = jnp.dot(h, wqkv_ref[...], preferred_element_type=jnp.float32)
    qkv = qkv + bqkv_ref[...]
    C = q_ref.shape[1]
    q_ref[...] = qkv[:, :C]
    k_ref[...] = qkv[:, C:2 * C]
    v_ref[...] = qkv[:, 2 * C:]


def _knn_body(pq_ref, pt_ref, ind_ref, d_ref, nsel):
    tq = pq_ref.shape[0]
    pq = pq_ref[...]
    pt = pt_ref[...]
    n = pt.shape[1]
    dot = jnp.dot(pq, pt, preferred_element_type=jnp.float32)
    sqk = jnp.sum(pt * pt, axis=0, keepdims=True)
    sqq = jnp.sum(pq * pq, axis=1, keepdims=True)
    iota_f = jax.lax.broadcasted_iota(jnp.int32, (tq, n), 1).astype(jnp.float32)
    d = sqq + sqk - 2.0 * dot
    big = jnp.float32(jnp.inf)
    nf = jnp.float32(n)
    cols = []
    idxf = None
    for k in range(nsel):
        if k == 0:
            d_ref[...] = d
            v = d
        else:
            v = jnp.where(iota_f == idxf, big, d_ref[...])
            d_ref[...] = v
        m = jnp.min(v, axis=1, keepdims=True)
        idxf = jnp.min(jnp.where(v == m, iota_f, nf), axis=1, keepdims=True)
        cols.append(idxf)
    pad = jnp.zeros((tq, 32 - nsel), jnp.float32)
    ind_ref[...] = jnp.concatenate(cols + [pad], axis=1).astype(jnp.int32)


def _p0_body(pg_ref, p8_ref, ew_ref, w1p_ref, b1p_ref, pr3_ref, st_ref):
    i = pl.program_id(0)
    r, c8 = p8_ref.shape
    k = pg_ref.shape[0] // r
    prj = jnp.dot(pg_ref[...], ew_ref[...],
                  preferred_element_type=jnp.float32)
    pri = jnp.dot(p8_ref[...], w1p_ref[...],
                  preferred_element_type=jnp.float32)
    pr = (prj.reshape(r, k, c8) - pri.reshape(r, 1, c8)
          + b1p_ref[...].reshape(1, 1, c8)).reshape(r * k, c8)
    pr3_ref[...] = pr

    @pl.when(i == 0)
    def _():
        st_ref[...] = jnp.zeros_like(st_ref)

    s = jnp.sum(pr, axis=0)
    ss = jnp.sum(pr * pr, axis=0)
    st_ref[...] += jnp.concatenate([s[None, :], ss[None, :]], axis=0)


def _pr_block(pr3, a3, wp2, bp2, r, k):
    pr3n = jnp.maximum(pr3 * a3[0:1, :] + a3[1:2, :], 0.0)
    pr = jnp.dot(pr3n.astype(jnp.bfloat16), wp2,
                 preferred_element_type=jnp.float32) + bp2
    return pr.reshape(r, k, -1)


def _p1_body(q_ref, kg_ref, pr3_ref, a3_ref, wp2_ref, bp2_ref, st_ref):
    i = pl.program_id(0)
    r, k, c = kg_ref.shape
    pr = _pr_block(pr3_ref[...], a3_ref[...], wp2_ref[...], bp2_ref[...],
                   r, k)
    w = q_ref[...].reshape(r, 1, c) - kg_ref[...] + pr

    @pl.when(i == 0)
    def _():
        st_ref[...] = jnp.zeros_like(st_ref)

    s = jnp.sum(w, axis=(0, 1))
    ss = jnp.sum(w * w, axis=(0, 1))
    st_ref[...] += jnp.concatenate([s[None, :], ss[None, :]], axis=0)


def _p2_body(q_ref, kg_ref, pr3_ref, a3_ref, wp2_ref, bp2_ref,
             a1_ref, ww1_ref, bw1_ref, u_ref, st_ref):
    i = pl.program_id(0)
    r, k, c = kg_ref.shape
    pr = _pr_block(pr3_ref[...], a3_ref[...], wp2_ref[...], bp2_ref[...],
                   r, k)
    w = q_ref[...].reshape(r, 1, c) - kg_ref[...] + pr
    w = jnp.maximum(w * a1_ref[0:1, :].reshape(1, 1, c)
                    + a1_ref[1:2, :].reshape(1, 1, c), 0.0)
    u = jnp.dot(w.reshape(r * k, c).astype(jnp.bfloat16), ww1_ref[...],
                preferred_element_type=jnp.float32) + bw1_ref[...]
    u_ref[...] = u

    @pl.when(i == 0)
    def _():
        st_ref[...] = jnp.zeros_like(st_ref)

    s = jnp.sum(u, axis=0)
    ss = jnp.sum(u * u, axis=0)
    st_ref[...] += jnp.concatenate([s[None, :], ss[None, :]], axis=0)


def _p3_body(u_ref, vg_ref, pr3_ref, a3_ref, wp2_ref, bp2_ref,
             a2_ref, ww2_ref, bw2_ref, t_ref, st_ref):
    i = pl.program_id(0)
    r, k, c = vg_ref.shape
    un = jnp.maximum(u_ref[...] * a2_ref[0:1, :] + a2_ref[1:2, :], 0.0)
    w2 = jnp.dot(un.astype(jnp.bfloat16), ww2_ref[...],
                 preferred_element_type=jnp.float32) + bw2_ref[...]
    w2 = w2.reshape(r, k, c)
    mx = jnp.max(w2, axis=1, keepdims=True)
    e = jnp.exp(w2 - mx)
    wgt = e / jnp.sum(e, axis=1, keepdims=True)
    pr = _pr_block(pr3_ref[...], a3_ref[...], wp2_ref[...], bp2_ref[...],
                   r, k)
    t = jnp.sum((vg_ref[...] + pr) * wgt, axis=1)
    t_ref[...] = t

    @pl.when(i == 0)
    def _():
        st_ref[...] = jnp.zeros_like(st_ref)

    s = jnp.sum(t, axis=0)
    ss = jnp.sum(t * t, axis=0)
    st_ref[...] += jnp.concatenate([s[None, :], ss[None, :]], axis=0)


def _p4_body(t_ref, at_ref, w3_ref, s_ref, st_ref):
    i = pl.program_id(0)
    tn = jnp.maximum(t_ref[...] * at_ref[0:1, :] + at_ref[1:2, :], 0.0)
    s = jnp.dot(tn.astype(jnp.bfloat16), w3_ref[...],
                preferred_element_type=jnp.float32)
    s_ref[...] = s

    @pl.when(i == 0)
    def _():
        st_ref[...] = jnp.zeros_like(st_ref)

    su = jnp.sum(s, axis=0)
    ss = jnp.sum(s * s, axis=0)
    st_ref[...] += jnp.concatenate([su[None, :], ss[None, :]], axis=0)


def _p5_body(s_ref, x_ref, as_ref, o_ref):
    o_ref[...] = jnp.maximum(
        s_ref[...] * as_ref[0:1, :] + as_ref[1:2, :] + x_ref[...], 0.0)


_SC_WINDOW = 128


def _sc_gather2(tab, pp, gi):
    m = gi.shape[0] * gi.shape[1]
    c = tab.shape[1]
    cp = pp.shape[1]
    w = _SC_WINDOW
    mesh = plsc.VectorSubcoreMesh(core_axis_name="core",
                                  subcore_axis_name="subcore")

    @functools.partial(
        pl.kernel,
        out_type=(jax.ShapeDtypeStruct((m, c), jnp.float32),
                  jax.ShapeDtypeStruct((m, cp), jnp.float32)),
        mesh=mesh,
        scratch_types=[],
    )
    def k(tab_hbm, pp_hbm, gi_hbm, tg_hbm, ppg_hbm):
        def body(i_vmem, to_vmem, ppo_vmem):
            pltpu.sync_copy(tab_hbm.at[i_vmem.at[0]], to_vmem)
            pltpu.sync_copy(pp_hbm.at[i_vmem.at[0]], ppo_vmem)

        pltpu.emit_pipeline(
            body,
            grid=(m // w,),
            in_specs=[pl.BlockSpec((1, w), lambda i: (i, 0))],
            out_specs=[pl.BlockSpec((w, c), lambda i: (i, 0)),
                       pl.BlockSpec((w, cp), lambda i: (i, 0))],
            core_axis_name=("core", "subcore"),
            dimension_semantics=(pltpu.PARALLEL,),
        )(gi_hbm, tg_hbm, ppg_hbm)

    return k(tab, pp, gi)


def _sc_gather1(tab, gi):
    m = gi.shape[0] * gi.shape[1]
    c = tab.shape[1]
    w = _SC_WINDOW
    mesh = plsc.VectorSubcoreMesh(core_axis_name="core",
                                  subcore_axis_name="subcore")

    @functools.partial(
        pl.kernel,
        out_type=jax.ShapeDtypeStruct((m, c), jnp.float32),
        mesh=mesh,
        scratch_types=[],
    )
    def k(tab_hbm, gi_hbm, tg_hbm):
        def body(i_vmem, to_vmem):
            pltpu.sync_copy(tab_hbm.at[i_vmem.at[0]], to_vmem)

        pltpu.emit_pipeline(
            body,
            grid=(m // w,),
            in_specs=[pl.BlockSpec((1, w), lambda i: (i, 0))],
            out_specs=[pl.BlockSpec((w, c), lambda i: (i, 0))],
            core_axis_name=("core", "subcore"),
            dimension_semantics=(pltpu.PARALLEL,),
        )(gi_hbm, tg_hbm)

    return k(tab, gi)


def _affine(st, g, b, count):
    m = st[0] / count
    var = jnp.maximum(st[1] / count - m * m, 0.0)
    scale = g / jnp.sqrt(var + _EPS)
    shift = b - m * scale
    return jnp.concatenate([scale[None, :], shift[None, :]], axis=0)


def kernel(x, p, knn_ind, params):
    B, N, C = x.shape
    K = knn_ind.shape[2]
    BN = B * N
    M = BN * K
    TM = 512
    TR = 128

    x2 = x.reshape(BN, C)
    p2 = p.reshape(BN, 3)
    p8 = jnp.concatenate([p2, jnp.zeros((BN, 5), jnp.float32)], axis=1)
    p128 = jnp.concatenate([p2, jnp.zeros((BN, 125), jnp.float32)], axis=1)
    pt = jnp.swapaxes(p, 1, 2)

    P = params
    w1t = P['W1'].T
    wqkv = jnp.concatenate([P['Wq'].T, P['Wk'].T, P['Wv'].T], axis=1)
    bqkv = jnp.concatenate([P['bq'], P['bk'], P['bv']])[None, :]
    w1p = jnp.zeros((8, 8), jnp.float32).at[:3, :3].set(P['Wp1'].T)
    ew = jnp.zeros((128, 8), jnp.float32).at[:3, :3].set(P['Wp1'].T)
    b1p = jnp.zeros((1, 8), jnp.float32).at[0, :3].set(P['bp1'])
    g3 = jnp.zeros((8,), jnp.float32).at[:3].set(P['bnp_g'])
    b3 = jnp.zeros((8,), jnp.float32).at[:3].set(P['bnp_b'])
    wp2 = jnp.zeros((8, C), jnp.float32).at[:3, :].set(
        P['Wp2'].T).astype(jnp.bfloat16)
    bp2 = P['bp2'][None, :]
    ww1 = P['Ww1'].T.astype(jnp.bfloat16)
    bw1 = P['bw1'][None, :]
    ww2 = P['Ww2'].T.astype(jnp.bfloat16)
    bw2 = P['bw2'][None, :]
    w3 = P['W3'].T.astype(jnp.bfloat16)

    cspec = pl.BlockSpec((2, C), lambda i: (0, 0))
    wspec = pl.BlockSpec((C, C), lambda i: (0, 0))
    rspec = pl.BlockSpec((TM, C), lambda i: (i, 0))

    h1, st1 = pl.pallas_call(
        _pre1_body,
        grid=(BN // TM,),
        in_specs=[rspec, wspec],
        out_specs=[rspec, cspec],
        out_shape=[jax.ShapeDtypeStruct((BN, C), jnp.float32),
                   jax.ShapeDtypeStruct((2, C), jnp.float32)],
    )(x2, w1t)
    aff1 = _affine(st1, P['bn1_g'], P['bn1_b'], BN)

    q, kt, vt = pl.pallas_call(
        _pre2_body,
        grid=(BN // TM,),
        in_specs=[rspec, cspec, pl.BlockSpec((C, 3 * C), lambda i: (0, 0)),
                  pl.BlockSpec((1, 3 * C), lambda i: (0, 0))],
        out_specs=[rspec, rspec, rspec],
        out_shape=[jax.ShapeDtypeStruct((BN, C), jnp.float32),
                   jax.ShapeDtypeStruct((BN, C), jnp.float32),
                   jax.ShapeDtypeStruct((BN, C), jnp.float32)],
    )(h1, aff1, wqkv, bqkv)

    NCPB = 2
    NCH = B * NCPB
    R = N // NCPB
    MC = R * K
    TQ = 256
    TP = 128

    chunks = []
    for c in range(NCH):
        b, off = c // NCPB, (c % NCPB) * R
        ind_c = pl.pallas_call(
            functools.partial(_knn_body, nsel=K + 1),
            grid=(R // TQ,),
            in_specs=[pl.BlockSpec((TQ, 3), lambda i: (i, 0)),
                      pl.BlockSpec((3, N), lambda i: (0, 0))],
            out_specs=pl.BlockSpec((TQ, 32), lambda i: (i, 0)),
            out_shape=jax.ShapeDtypeStruct((R, 32), jnp.int32),
            scratch_shapes=[pltpu.VMEM((TQ, N), jnp.float32)],
        )(p[b, off:off + R], pt[b])
        gi_c = (ind_c[:, 1:K + 1] + b * N).reshape(MC // _SC_WINDOW,
                                                   _SC_WINDOW)
        chunks.append((ind_c, gi_c))

    gath = []
    for c in range(NCH):
        _, gi_c = chunks[c]
        kg_c, pg_c = _sc_gather2(kt, p128, gi_c)
        gath.append((kg_c.reshape(R, K, C), pg_c))

    pr3s, st3s = [], []
    for c in range(NCH):
        base = (c // NCPB) * N + (c % NCPB) * R
        pr3_c, st3_c = pl.pallas_call(
            _p0_body,
            grid=(R // TP,),
            in_specs=[pl.BlockSpec((TP * K, 128), lambda i: (i, 0)),
                      pl.BlockSpec((TP, 8),
                                   functools.partial(
                                       lambda base, i: (i + base, 0),
                                       base // TP)),
                      pl.BlockSpec((128, 8), lambda i: (0, 0)),
                      pl.BlockSpec((8, 8), lambda i: (0, 0)),
                      pl.BlockSpec((1, 8), lambda i: (0, 0))],
            out_specs=[pl.BlockSpec((TP * K, 8), lambda i: (i, 0)),
                       pl.BlockSpec((2, 8), lambda i: (0, 0))],
            out_shape=[jax.ShapeDtypeStruct((MC, 8), jnp.float32),
                       jax.ShapeDtypeStruct((2, 8), jnp.float32)],
        )(gath[c][1], p8, ew, w1p, b1p)
        pr3s.append(pr3_c)
        st3s.append(st3_c)
    aff3 = _affine(sum(st3s[1:], st3s[0]), g3, b3, M)

    kgspec = pl.BlockSpec((TR, K, C), lambda i: (i, 0, 0))
    vgspec = pl.BlockSpec((TR, K, C), lambda i: (i, 0, 0))
    pr3spec = pl.BlockSpec((TR * K, 8), lambda i: (i, 0))
    a3spec = pl.BlockSpec((2, 8), lambda i: (0, 0))
    wp2spec = pl.BlockSpec((8, C), lambda i: (0, 0))
    bspec = pl.BlockSpec((1, C), lambda i: (0, 0))
    uspec = pl.BlockSpec((TR * K, C), lambda i: (i, 0))

    def _qspec(c):
        base = ((c // NCPB) * N + (c % NCPB) * R) // TR
        return pl.BlockSpec(
            (TR, C), functools.partial(lambda b0, i: (i + b0, 0), base))

    stw1 = None
    for c in range(NCH):
        st_c = pl.pallas_call(
            _p1_body,
            grid=(R // TR,),
            in_specs=[_qspec(c), kgspec, pr3spec, a3spec, wp2spec, bspec],
            out_specs=cspec,
            out_shape=jax.ShapeDtypeStruct((2, C), jnp.float32),
        )(q, gath[c][0], pr3s[c], aff3, wp2, bp2)
        stw1 = st_c if stw1 is None else stw1 + st_c
    affw1 = _affine(stw1, P['bnw1_g'], P['bnw1_b'], M)

    vgs = [_sc_gather1(vt, chunks[c][1]).reshape(R, K, C)
           for c in range(NCH)]

    us, stw2 = [], None
    for c in range(NCH):
        u_c, st_c = pl.pallas_call(
            _p2_body,
            grid=(R // TR,),
            in_specs=[_qspec(c), kgspec, pr3spec, a3spec, wp2spec, bspec,
                      cspec, wspec, bspec],
            out_specs=[uspec, cspec],
            out_shape=[jax.ShapeDtypeStruct((MC, C), jnp.float32),
                       jax.ShapeDtypeStruct((2, C), jnp.float32)],
        )(q, gath[c][0], pr3s[c], aff3, wp2, bp2, affw1, ww1, bw1)
        us.append(u_c)
        stw2 = st_c if stw2 is None else stw2 + st_c
    affw2 = _affine(stw2, P['bnw2_g'], P['bnw2_b'], M)

    ts, stt = [], None
    for c in range(NCH):
        t_c, st_c = pl.pallas_call(
            _p3_body,
            grid=(R // TR,),
            in_specs=[uspec, vgspec, pr3spec, a3spec, wp2spec, bspec,
                      cspec, wspec, bspec],
            out_specs=[pl.BlockSpec((TR, C), lambda i: (i, 0)), cspec],
            out_shape=[jax.ShapeDtypeStruct((R, C), jnp.float32),
                       jax.ShapeDtypeStruct((2, C), jnp.float32)],
        )(us[c], vgs[c], pr3s[c], aff3, wp2, bp2, affw2, ww2, bw2)
        ts.append(t_c)
        stt = st_c if stt is None else stt + st_c
    t = jnp.concatenate(ts, axis=0)
    afft = _affine(stt, P['bn2_g'], P['bn2_b'], BN)

    s, sts = pl.pallas_call(
        _p4_body,
        grid=(BN // TM,),
        in_specs=[rspec, cspec, wspec],
        out_specs=[rspec, cspec],
        out_shape=[jax.ShapeDtypeStruct((BN, C), jnp.float32),
                   jax.ShapeDtypeStruct((2, C), jnp.float32)],
    )(t, afft, w3)
    affs = _affine(sts, P['bn3_g'], P['bn3_b'], BN)

    out = pl.pallas_call(
        _p5_body,
        grid=(BN // TM,),
        in_specs=[rspec, rspec, cspec],
        out_specs=rspec,
        out_shape=jax.ShapeDtypeStruct((BN, C), jnp.float32),
    )(s, x2, affs)

    ind = jnp.stack(
        [jnp.concatenate([chunks[b * NCPB + j][0][:, 1:K + 1]
                          for j in range(NCPB)], axis=0) for b in range(B)],
        axis=0)
    return (out.reshape(B, N, C), p, ind)
